# baseline (speedup 1.0000x reference)
import functools

import jax
import jax.numpy as jnp
from jax import lax
from jax.experimental import pallas as pl
from jax.experimental.pallas import tpu as pltpu
from jax.experimental.pallas import tpu_sc as plsc

B = 32
N = 20000
CIN = 10
COUT = 9
SCALE = 1.0 / 640.0

NC = 2
NS = 16
NW = NC * NS
BPW = (B * N) // NW
CHUNK = 2000
GROUPS = CHUNK // 16
NCHUNKS = BPW // CHUNK

_mesh = plsc.VectorSubcoreMesh(core_axis_name="c", subcore_axis_name="s")


@functools.partial(
    pl.kernel,
    mesh=_mesh,
    out_type=jax.ShapeDtypeStruct((B * N * COUT,), jnp.float32),
    compiler_params=pltpu.CompilerParams(needs_layout_passes=False),
    scratch_types=[
        pltpu.VMEM((CHUNK * CIN,), jnp.float32),
        pltpu.VMEM((CHUNK * CIN,), jnp.float32),
        pltpu.VMEM((CHUNK * COUT,), jnp.float32),
        pltpu.VMEM((CHUNK * COUT,), jnp.float32),
        pltpu.SemaphoreType.DMA,
        pltpu.SemaphoreType.DMA,
        pltpu.SemaphoreType.DMA,
        pltpu.SemaphoreType.DMA,
    ],
)
def _adapter(pred_hbm, out_hbm, inb0, inb1, outb0, outb1,
             sem_i0, sem_i1, sem_o0, sem_o1):
    cid = lax.axis_index("c")
    sid = lax.axis_index("s")
    wid = sid * NC + cid
    in_base = wid * (BPW * CIN)
    out_base = wid * (BPW * COUT)

    inb = (inb0, inb1)
    outb = (outb0, outb1)
    sem_i = (sem_i0, sem_i1)
    sem_o = (sem_o0, sem_o1)

    iota = lax.iota(jnp.int32, 16)
    idx_in = [iota * CIN + c for c in range(CIN)]
    idx_out = [iota * COUT + c for c in range(COUT)]

    def start_in(k):
        return pltpu.async_copy(
            pred_hbm.at[pl.ds(in_base + k * (CHUNK * CIN), CHUNK * CIN)],
            inb[k % 2], sem_i[k % 2])

    def start_out(k):
        return pltpu.async_copy(
            outb[k % 2],
            out_hbm.at[pl.ds(out_base + k * (CHUNK * COUT), CHUNK * COUT)],
            sem_o[k % 2])

    in_cp = [None, None]
    out_cp = [None, None]
    in_cp[0] = start_in(0)

    for k in range(NCHUNKS):
        b = k % 2
        if k + 1 < NCHUNKS:
            in_cp[1 - b] = start_in(k + 1)
        in_cp[b].wait()
        if out_cp[b] is not None:
            out_cp[b].wait()
        src = inb[b]
        dst = outb[b]

        @plsc.parallel_loop(0, GROUPS, unroll=4)
        def group_body(g):
            gi = g * (16 * CIN)
            go = g * (16 * COUT)
            x1 = plsc.load_gather(src, [idx_in[0] + gi])
            y1 = plsc.load_gather(src, [idx_in[1] + gi])
            x2 = plsc.load_gather(src, [idx_in[2] + gi])
            y2 = plsc.load_gather(src, [idx_in[3] + gi])
            lab = plsc.load_gather(src, [idx_in[4] + gi])
            a0 = plsc.load_gather(src, [idx_in[6] + gi])
            a1 = plsc.load_gather(src, [idx_in[7] + gi])
            a2 = plsc.load_gather(src, [idx_in[8] + gi])
            a3 = plsc.load_gather(src, [idx_in[9] + gi])
            plsc.store_scatter(dst, [idx_out[0] + go], x1 * SCALE)
            plsc.store_scatter(dst, [idx_out[1] + go], y1 * SCALE)
            plsc.store_scatter(dst, [idx_out[2] + go], (x2 - x1) * SCALE)
            plsc.store_scatter(dst, [idx_out[3] + go], (y2 - y1) * SCALE)
            plsc.store_scatter(dst, [idx_out[4] + go], a0)
            plsc.store_scatter(dst, [idx_out[5] + go], a1)
            plsc.store_scatter(dst, [idx_out[6] + go], a2)
            plsc.store_scatter(dst, [idx_out[7] + go], a3)
            plsc.store_scatter(dst, [idx_out[8] + go], lab)

        out_cp[b] = start_out(k)

    out_cp[0].wait()
    out_cp[1].wait()


def kernel(predictions):
    flat = predictions.reshape(-1)
    out = _adapter(flat)
    return out.reshape(B, N, COUT)


# speedup vs baseline: 20.3430x; 20.3430x over previous
import functools

import jax
import jax.numpy as jnp
from jax import lax
from jax.experimental import pallas as pl
from jax.experimental.pallas import tpu as pltpu
from jax.experimental.pallas import tpu_sc as plsc

B = 32
N = 20000
CIN = 10
COUT = 9
SCALE = 1.0 / 640.0

NW = 32
TCOLS = 157
FULLCOLS = TCOLS - 1
ROUNDS = 5

IN_PLANES = [0, 1, 2, 3, 4, 6, 7, 8, 9]
OUT_SRC = [0, 1, 2, 3, 6, 7, 8, 9, 4]
NBUF = len(IN_PLANES)

_mesh = plsc.VectorSubcoreMesh(core_axis_name="c", subcore_axis_name="s")


@functools.partial(
    pl.kernel,
    mesh=_mesh,
    out_type=jax.ShapeDtypeStruct((COUT, B, N), jnp.float32),
    compiler_params=pltpu.CompilerParams(needs_layout_passes=False),
    scratch_types=(
        [pltpu.VMEM((B, 128), jnp.float32) for _ in range(NBUF)]
        + [pltpu.SemaphoreType.DMA, pltpu.SemaphoreType.DMA]
    ),
)
def _adapter(pred_hbm, out_hbm, *refs):
    buf = dict(zip(IN_PLANES, refs[:NBUF]))
    sem_i, sem_o = refs[NBUF], refs[NBUF + 1]

    cid = lax.axis_index("c")
    sid = lax.axis_index("s")
    wid = sid * 2 + cid

    def compute():
        @plsc.parallel_loop(0, B * 8, unroll=4)
        def body(i):
            r = i // 8
            j = (i % 8) * 16
            x1 = buf[0][r, pl.ds(j, 16)]
            y1 = buf[1][r, pl.ds(j, 16)]
            x2 = buf[2][r, pl.ds(j, 16)]
            y2 = buf[3][r, pl.ds(j, 16)]
            buf[2][r, pl.ds(j, 16)] = (x2 - x1) * SCALE
            buf[3][r, pl.ds(j, 16)] = (y2 - y1) * SCALE
            buf[0][r, pl.ds(j, 16)] = x1 * SCALE
            buf[1][r, pl.ds(j, 16)] = y1 * SCALE

    def do_col(tc):
        col = tc * 128
        cps = [pltpu.async_copy(
            pred_hbm.at[c, :, pl.ds(col, 128)], buf[c], sem_i)
            for c in IN_PLANES]
        for cp in cps:
            cp.wait()
        compute()
        cps = [pltpu.async_copy(
            buf[OUT_SRC[o]], out_hbm.at[o, :, pl.ds(col, 128)], sem_o)
            for o in range(COUT)]
        for cp in cps:
            cp.wait()

    for i in range(ROUNDS):
        tc = wid + NW * i
        if (i + 1) * NW <= FULLCOLS:
            do_col(tc)
        else:
            @pl.when(tc < FULLCOLS)
            def _masked():
                do_col(tc)


def _tail_body(x_ref, alias_ref, o_ref):
    x = x_ref[...]
    bb = x[0:4]
    wh = bb[2:4] - bb[0:2]
    o_ref[...] = jnp.concatenate(
        [bb[0:2] * SCALE, wh * SCALE, x[6:10], x[4:5]], axis=0)


_tail_call = pl.pallas_call(
    _tail_body,
    out_shape=jax.ShapeDtypeStruct((COUT, B, N), jnp.float32),
    grid=(1,),
    in_specs=[
        pl.BlockSpec((CIN, B, 128), lambda i: (0, 0, FULLCOLS)),
        pl.BlockSpec(memory_space=pl.ANY),
    ],
    out_specs=pl.BlockSpec((COUT, B, 128), lambda i: (0, 0, FULLCOLS)),
    input_output_aliases={1: 0},
)


def kernel(predictions):
    planar = jnp.transpose(predictions, (2, 0, 1))
    main = _adapter(planar)
    full = _tail_call(planar, main)
    return jnp.transpose(full, (1, 2, 0))


# speedup vs baseline: 22.9189x; 1.1266x over previous
import functools

import jax
import jax.numpy as jnp
from jax import lax
from jax.experimental import pallas as pl
from jax.experimental.pallas import tpu as pltpu
from jax.experimental.pallas import tpu_sc as plsc

B = 32
N = 20000
CIN = 10
COUT = 9
SCALE = 1.0 / 640.0

NW = 32
TCOLS = 157
FULLCOLS = TCOLS - 1
ROUNDS = 5

IN_PLANES = [0, 1, 2, 3, 4, 6, 7, 8, 9]
OUT_SRC = [0, 1, 2, 3, 6, 7, 8, 9, 4]
NBUF = len(IN_PLANES)

_mesh = plsc.VectorSubcoreMesh(core_axis_name="c", subcore_axis_name="s")


SETS = 3
LASTW = FULLCOLS - (ROUNDS - 1) * NW


@functools.partial(
    pl.kernel,
    mesh=_mesh,
    out_type=jax.ShapeDtypeStruct((COUT, B, N), jnp.float32),
    compiler_params=pltpu.CompilerParams(needs_layout_passes=False),
    scratch_types=(
        [pltpu.VMEM((B, 128), jnp.float32) for _ in range(SETS * NBUF)]
        + [pltpu.SemaphoreType.DMA for _ in range(2 * SETS)]
    ),
)
def _adapter(pred_hbm, out_hbm, *refs):
    bufs = [dict(zip(IN_PLANES, refs[s * NBUF:(s + 1) * NBUF]))
            for s in range(SETS)]
    sem_i = refs[SETS * NBUF:SETS * NBUF + SETS]
    sem_o = refs[SETS * NBUF + SETS:SETS * NBUF + 2 * SETS]

    cid = lax.axis_index("c")
    sid = lax.axis_index("s")
    wid = sid * 2 + cid

    def compute(buf):
        @plsc.parallel_loop(0, B * 8, unroll=4)
        def body(i):
            r = i // 8
            j = (i % 8) * 16
            x1 = buf[0][r, pl.ds(j, 16)]
            y1 = buf[1][r, pl.ds(j, 16)]
            x2 = buf[2][r, pl.ds(j, 16)]
            y2 = buf[3][r, pl.ds(j, 16)]
            buf[2][r, pl.ds(j, 16)] = (x2 - x1) * SCALE
            buf[3][r, pl.ds(j, 16)] = (y2 - y1) * SCALE
            buf[0][r, pl.ds(j, 16)] = x1 * SCALE
            buf[1][r, pl.ds(j, 16)] = y1 * SCALE

    def in_copies(k):
        s = k % SETS
        col = (wid + NW * k) * 128
        return [pltpu.make_async_copy(
            pred_hbm.at[c, :, pl.ds(col, 128)], bufs[s][c], sem_i[s])
            for c in IN_PLANES]

    def out_copies(k):
        s = k % SETS
        col = (wid + NW * k) * 128
        return [pltpu.make_async_copy(
            bufs[s][OUT_SRC[o]], out_hbm.at[o, :, pl.ds(col, 128)],
            sem_o[s])
            for o in range(COUT)]

    def start(cps):
        for cp in cps:
            cp.start()

    def wait(cps):
        for cp in cps:
            cp.wait()

    start(in_copies(0))
    start(in_copies(1))

    for k in range(ROUNDS - 1):
        wait(in_copies(k))
        compute(bufs[k % SETS])
        start(out_copies(k))
        nk = k + 2
        if nk <= ROUNDS - 1:
            pk = nk - SETS
            if pk >= 0:
                wait(out_copies(pk))
            if nk < ROUNDS - 1:
                start(in_copies(nk))
            else:
                @pl.when(wid < LASTW)
                def _issue_last():
                    start(in_copies(nk))

    @pl.when(wid < LASTW)
    def _last_round():
        k = ROUNDS - 1
        wait(in_copies(k))
        compute(bufs[k % SETS])
        start(out_copies(k))

    wait(out_copies(ROUNDS - 3))
    wait(out_copies(ROUNDS - 2))

    @pl.when(wid < LASTW)
    def _drain_last():
        wait(out_copies(ROUNDS - 1))


def _tail_body(x_ref, alias_ref, o_ref):
    x = x_ref[...]
    bb = x[0:4]
    wh = bb[2:4] - bb[0:2]
    o_ref[...] = jnp.concatenate(
        [bb[0:2] * SCALE, wh * SCALE, x[6:10], x[4:5]], axis=0)


_tail_call = pl.pallas_call(
    _tail_body,
    out_shape=jax.ShapeDtypeStruct((COUT, B, N), jnp.float32),
    grid=(1,),
    in_specs=[
        pl.BlockSpec((CIN, B, 128), lambda i: (0, 0, FULLCOLS)),
        pl.BlockSpec(memory_space=pl.ANY),
    ],
    out_specs=pl.BlockSpec((COUT, B, 128), lambda i: (0, 0, FULLCOLS)),
    input_output_aliases={1: 0},
)


def kernel(predictions):
    planar = jnp.transpose(predictions, (2, 0, 1))
    main = _adapter(planar)
    full = _tail_call(planar, main)
    return jnp.transpose(full, (1, 2, 0))


# speedup vs baseline: 23.1283x; 1.0091x over previous
import functools

import jax
import jax.numpy as jnp
from jax import lax
from jax.experimental import pallas as pl
from jax.experimental.pallas import tpu as pltpu
from jax.experimental.pallas import tpu_sc as plsc

B = 32
N = 20000
CIN = 10
COUT = 9
SCALE = 1.0 / 640.0

NW = 32
TCOLS = 157
FULLCOLS = TCOLS - 1
ROUNDS = 5

IN_PLANES = [0, 1, 2, 3, 4, 6, 7, 8, 9]
OUT_SRC = [0, 1, 2, 3, 6, 7, 8, 9, 4]
NBUF = len(IN_PLANES)

_mesh = plsc.VectorSubcoreMesh(core_axis_name="c", subcore_axis_name="s")


SETS = 3
LASTW = FULLCOLS - (ROUNDS - 1) * NW


@functools.partial(
    pl.kernel,
    mesh=_mesh,
    out_type=jax.ShapeDtypeStruct((COUT, B, N), jnp.float32),
    compiler_params=pltpu.CompilerParams(needs_layout_passes=False),
    scratch_types=(
        [pltpu.VMEM((B, 128), jnp.float32) for _ in range(SETS * NBUF)]
        + [pltpu.SemaphoreType.DMA for _ in range(2 * SETS)]
    ),
)
def _adapter(pred_hbm, out_hbm, *refs):
    bufs = [dict(zip(IN_PLANES, refs[s * NBUF:(s + 1) * NBUF]))
            for s in range(SETS)]
    sem_i = refs[SETS * NBUF:SETS * NBUF + SETS]
    sem_o = refs[SETS * NBUF + SETS:SETS * NBUF + 2 * SETS]

    cid = lax.axis_index("c")
    sid = lax.axis_index("s")
    wid = sid * 2 + cid

    def compute(buf):
        @plsc.parallel_loop(0, B * 8, unroll=2)
        def body(i):
            r = i // 8
            j = (i % 8) * 16
            x1 = buf[0][r, pl.ds(j, 16)]
            y1 = buf[1][r, pl.ds(j, 16)]
            x2 = buf[2][r, pl.ds(j, 16)]
            y2 = buf[3][r, pl.ds(j, 16)]
            buf[2][r, pl.ds(j, 16)] = (x2 - x1) * SCALE
            buf[3][r, pl.ds(j, 16)] = (y2 - y1) * SCALE
            buf[0][r, pl.ds(j, 16)] = x1 * SCALE
            buf[1][r, pl.ds(j, 16)] = y1 * SCALE

    def col_of(k):
        return (wid + NW * k) * 128

    def in_copies(k):
        s = k % SETS
        col = col_of(k)
        return [pltpu.make_async_copy(
            pred_hbm.at[c, :, pl.ds(col, 128)], bufs[s][c], sem_i[s])
            for c in IN_PLANES]

    def out_copies(k):
        s = k % SETS
        col = col_of(k)
        return [pltpu.make_async_copy(
            bufs[s][OUT_SRC[o]], out_hbm.at[o, :, pl.ds(col, 128)],
            sem_o[s])
            for o in range(COUT)]

    def start(cps):
        for cp in cps:
            cp.start()

    def wait(cps):
        for cp in cps:
            cp.wait()

    start(in_copies(0))
    start(in_copies(1))

    for k in range(ROUNDS - 1):
        wait(in_copies(k))
        compute(bufs[k % SETS])
        start(out_copies(k))
        nk = k + 2
        if nk <= ROUNDS - 1:
            pk = nk - SETS
            if pk >= 0:
                wait(out_copies(pk))
            if nk < ROUNDS - 1:
                start(in_copies(nk))
            else:
                @pl.when(wid < LASTW)
                def _issue_last():
                    start(in_copies(nk))

    @pl.when(wid < LASTW)
    def _last_round():
        k = ROUNDS - 1
        wait(in_copies(k))
        compute(bufs[k % SETS])
        start(out_copies(k))

    wait(out_copies(ROUNDS - 3))
    wait(out_copies(ROUNDS - 2))

    @pl.when(wid < LASTW)
    def _drain_last():
        wait(out_copies(ROUNDS - 1))


def _tail_body(x_ref, alias_ref, o_ref):
    x = x_ref[...]
    bb = x[0:4]
    wh = bb[2:4] - bb[0:2]
    o_ref[...] = jnp.concatenate(
        [bb[0:2] * SCALE, wh * SCALE, x[6:10], x[4:5]], axis=0)


_tail_call = pl.pallas_call(
    _tail_body,
    out_shape=jax.ShapeDtypeStruct((COUT, B, N), jnp.float32),
    grid=(1,),
    in_specs=[
        pl.BlockSpec((CIN, B, 128), lambda i: (0, 0, FULLCOLS)),
        pl.BlockSpec(memory_space=pl.ANY),
    ],
    out_specs=pl.BlockSpec((COUT, B, 128), lambda i: (0, 0, FULLCOLS)),
    input_output_aliases={1: 0},
)


def kernel(predictions):
    planar = jnp.transpose(predictions, (2, 0, 1))
    main = _adapter(planar)
    full = _tail_call(planar, main)
    return jnp.transpose(full, (1, 2, 0))


# speedup vs baseline: 23.2077x; 1.0034x over previous
import functools

import jax
import jax.numpy as jnp
from jax import lax
from jax.experimental import pallas as pl
from jax.experimental.pallas import tpu as pltpu
from jax.experimental.pallas import tpu_sc as plsc

B = 32
N = 20000
CIN = 10
COUT = 9
SCALE = 1.0 / 640.0

NW = 32
TCOLS = 157
FULLCOLS = TCOLS - 1
ROUNDS = 5

IN_PLANES = [0, 1, 2, 3, 4, 6, 7, 8, 9]
OUT_SRC = [0, 1, 2, 3, 6, 7, 8, 9, 4]
NBUF = len(IN_PLANES)

_mesh = plsc.VectorSubcoreMesh(core_axis_name="c", subcore_axis_name="s")


SETS = 3
LASTW = FULLCOLS - (ROUNDS - 1) * NW


@functools.partial(
    pl.kernel,
    mesh=_mesh,
    out_type=jax.ShapeDtypeStruct((COUT, B, N), jnp.float32),
    compiler_params=pltpu.CompilerParams(needs_layout_passes=False),
    scratch_types=(
        [pltpu.VMEM((16, 256), jnp.float32) for _ in range(SETS * NBUF)]
        + [pltpu.SemaphoreType.DMA for _ in range(2 * SETS)]
    ),
)
def _adapter(pred_hbm, out_hbm, *refs):
    bufs = [dict(zip(IN_PLANES, refs[s * NBUF:(s + 1) * NBUF]))
            for s in range(SETS)]
    sem_i = refs[SETS * NBUF:SETS * NBUF + SETS]
    sem_o = refs[SETS * NBUF + SETS:SETS * NBUF + 2 * SETS]

    cid = lax.axis_index("c")
    sid = lax.axis_index("s")
    wid = sid * 2 + cid

    def compute(buf):
        @plsc.parallel_loop(0, 16 * 16, unroll=2)
        def body(i):
            r = i // 16
            j = (i % 16) * 16
            x1 = buf[0][r, pl.ds(j, 16)]
            y1 = buf[1][r, pl.ds(j, 16)]
            x2 = buf[2][r, pl.ds(j, 16)]
            y2 = buf[3][r, pl.ds(j, 16)]
            buf[2][r, pl.ds(j, 16)] = (x2 - x1) * SCALE
            buf[3][r, pl.ds(j, 16)] = (y2 - y1) * SCALE
            buf[0][r, pl.ds(j, 16)] = x1 * SCALE
            buf[1][r, pl.ds(j, 16)] = y1 * SCALE

    def rowcol_of(k):
        u = wid + NW * k
        return (u % 2) * 16, (u // 2) * 256

    def in_copies(k):
        s = k % SETS
        row, col = rowcol_of(k)
        return [pltpu.make_async_copy(
            pred_hbm.at[c, pl.ds(row, 16), pl.ds(col, 256)],
            bufs[s][c], sem_i[s])
            for c in IN_PLANES]

    def out_copies(k):
        s = k % SETS
        row, col = rowcol_of(k)
        return [pltpu.make_async_copy(
            bufs[s][OUT_SRC[o]],
            out_hbm.at[o, pl.ds(row, 16), pl.ds(col, 256)], sem_o[s])
            for o in range(COUT)]

    def start(cps):
        for cp in cps:
            cp.start()

    def wait(cps):
        for cp in cps:
            cp.wait()

    start(in_copies(0))
    start(in_copies(1))

    for k in range(ROUNDS - 1):
        wait(in_copies(k))
        compute(bufs[k % SETS])
        start(out_copies(k))
        nk = k + 2
        if nk <= ROUNDS - 1:
            pk = nk - SETS
            if pk >= 0:
                wait(out_copies(pk))
            if nk < ROUNDS - 1:
                start(in_copies(nk))
            else:
                @pl.when(wid < LASTW)
                def _issue_last():
                    start(in_copies(nk))

    @pl.when(wid < LASTW)
    def _last_round():
        k = ROUNDS - 1
        wait(in_copies(k))
        compute(bufs[k % SETS])
        start(out_copies(k))

    wait(out_copies(ROUNDS - 3))
    wait(out_copies(ROUNDS - 2))

    @pl.when(wid < LASTW)
    def _drain_last():
        wait(out_copies(ROUNDS - 1))


def _tail_body(x_ref, alias_ref, o_ref):
    x = x_ref[...]
    bb = x[0:4]
    wh = bb[2:4] - bb[0:2]
    o_ref[...] = jnp.concatenate(
        [bb[0:2] * SCALE, wh * SCALE, x[6:10], x[4:5]], axis=0)


_tail_call = pl.pallas_call(
    _tail_body,
    out_shape=jax.ShapeDtypeStruct((COUT, B, N), jnp.float32),
    grid=(1,),
    in_specs=[
        pl.BlockSpec((CIN, B, 128), lambda i: (0, 0, FULLCOLS)),
        pl.BlockSpec(memory_space=pl.ANY),
    ],
    out_specs=pl.BlockSpec((COUT, B, 128), lambda i: (0, 0, FULLCOLS)),
    input_output_aliases={1: 0},
)


def kernel(predictions):
    planar = jnp.transpose(predictions, (2, 0, 1))
    main = _adapter(planar)
    full = _tail_call(planar, main)
    return jnp.transpose(full, (1, 2, 0))
